# baseline (speedup 1.0000x reference)
import jax
import jax.numpy as jnp
from jax import lax
from jax.experimental import pallas as pl
from jax.experimental.pallas import tpu as pltpu
from jax.experimental.pallas import tpu_sc as plsc

D_SUB = 256
D_MODEL = 1024
ROWS = 1000
NC, NS = 2, 16
NW = NC * NS
B_TOTAL = 4 * 8192
PER_W = B_TOTAL // NW
C = 32
NCHUNK = PER_W // C
L = 16


def _sc_body(icomb_hbm, ctab, out, icomb_w, b0, b1, g0, g1, w0, w1):
    wid = lax.axis_index("s") * NC + lax.axis_index("c")
    base_w = wid * PER_W
    bufs = (b0, b1)
    gsems = (g0, g1)
    wsems = (w0, w1)

    pltpu.sync_copy(icomb_hbm.at[pl.ds(4 * base_w, 4 * PER_W)], icomb_w)

    def gather(p, c):
        return pltpu.make_async_copy(
            ctab.at[icomb_w.at[pl.ds(4 * C * c, 4 * C)]], bufs[p], gsems[p])

    def write(p, c):
        return pltpu.make_async_copy(
            bufs[p], out.at[pl.ds(4 * (base_w + c * C), 4 * C), :], wsems[p])

    gather(0, 0).start()
    for c in range(NCHUNK):
        p = c % 2
        q = 1 - p
        if c + 1 < NCHUNK:
            if c >= 1:
                write(q, c - 1).wait()
            gather(q, c + 1).start()
        gather(p, c).wait()
        write(p, c).start()
    write(1, NCHUNK - 1).wait()


@jax.jit
def _lookup(tpos, cpos, epos, dpos, tt, ct, et, dt_):
    ctab = jnp.concatenate([tt, ct, et, dt_], axis=0)
    icomb = jnp.stack(
        [tpos.reshape(-1), cpos.reshape(-1) + ROWS,
         epos.reshape(-1) + 2 * ROWS, dpos.reshape(-1) + 3 * ROWS],
        axis=-1).reshape(-1)
    mesh = plsc.VectorSubcoreMesh(
        core_axis_name="c", subcore_axis_name="s",
        num_cores=NC, num_subcores=NS)
    scratch = (
        [pltpu.VMEM((4 * PER_W,), jnp.int32)]
        + [pltpu.VMEM((4 * C, D_SUB), jnp.float32) for _ in range(2)]
        + [pltpu.SemaphoreType.DMA for _ in range(4)]
    )
    f = pl.kernel(
        _sc_body,
        out_type=jax.ShapeDtypeStruct((4 * B_TOTAL, D_SUB), jnp.float32),
        mesh=mesh,
        scratch_types=scratch,
    )
    return f(icomb, ctab)


def kernel(temporal_pos, causal_depth, epistemic_pos, deontic_pos,
           temporal_table, causal_table, epistemic_table, deontic_table):
    b, s = temporal_pos.shape
    out = _lookup(temporal_pos, causal_depth, epistemic_pos, deontic_pos,
                  temporal_table, causal_table, epistemic_table, deontic_table)
    return out.reshape(b, s, D_MODEL)


# speedup vs baseline: 2.3250x; 2.3250x over previous
import jax
import jax.numpy as jnp
from jax import lax
from jax.experimental import pallas as pl
from jax.experimental.pallas import tpu as pltpu
from jax.experimental.pallas import tpu_sc as plsc

D_SUB = 256
D_MODEL = 1024
NC, NS = 2, 16
NW = NC * NS
B_TOTAL = 4 * 8192
PER_W = B_TOTAL // NW
C = 32
NCHUNK = PER_W // C


def _sc_body(tpos, cpos, epos, dpos, tt, ct, et, dt_, out, *scr):
    wid = lax.axis_index("s") * NC + lax.axis_index("c")
    base_w = wid * PER_W
    pos_refs = (tpos, cpos, epos, dpos)
    tab_refs = (tt, ct, et, dt_)
    idx_all = scr[0:4]
    bufs = tuple((scr[4 + 2 * t], scr[5 + 2 * t]) for t in range(4))
    gsems = tuple((scr[12 + 2 * t], scr[13 + 2 * t]) for t in range(4))
    wsems = tuple((scr[20 + 2 * t], scr[21 + 2 * t]) for t in range(4))

    for t in range(4):
        pltpu.sync_copy(pos_refs[t].at[pl.ds(base_w, PER_W)], idx_all[t])

    def gather(t, p, c):
        return pltpu.make_async_copy(
            tab_refs[t].at[idx_all[t].at[pl.ds(c * C, C)]],
            bufs[t][p], gsems[t][p])

    def write(t, p, c):
        return pltpu.make_async_copy(
            bufs[t][p],
            out.at[pl.ds(base_w + c * C, C), pl.ds(t * D_SUB, D_SUB)],
            wsems[t][p])

    def steady(c, p):
        for t in range(4):
            gather(t, p, c).wait()
            write(t, p, c).start()
        for t in range(4):
            write(t, 1 - p, c - 1).wait()
            gather(t, 1 - p, c + 1).start()

    for t in range(4):
        gather(t, 0, 0).start()
    for t in range(4):
        gather(t, 0, 0).wait()
        write(t, 0, 0).start()
    for t in range(4):
        gather(t, 1, 1).start()

    @pl.loop(1, NCHUNK - 2, step=2)
    def _main(c):
        steady(c, 1)
        steady(c + 1, 0)

    for t in range(4):
        gather(t, 1, NCHUNK - 1).wait()
        write(t, 1, NCHUNK - 1).start()
    for t in range(4):
        write(t, 0, NCHUNK - 2).wait()
        write(t, 1, NCHUNK - 1).wait()


@jax.jit
def _lookup(tpos, cpos, epos, dpos, tt, ct, et, dt_):
    mesh = plsc.VectorSubcoreMesh(
        core_axis_name="c", subcore_axis_name="s",
        num_cores=NC, num_subcores=NS)
    scratch = (
        [pltpu.VMEM((PER_W,), jnp.int32) for _ in range(4)]
        + [pltpu.VMEM((C, D_SUB), jnp.float32) for _ in range(8)]
        + [pltpu.SemaphoreType.DMA for _ in range(16)]
    )
    f = pl.kernel(
        _sc_body,
        out_type=jax.ShapeDtypeStruct((B_TOTAL, D_MODEL), jnp.float32),
        mesh=mesh,
        scratch_types=scratch,
    )
    return f(tpos.reshape(-1), cpos.reshape(-1), epos.reshape(-1),
             dpos.reshape(-1), tt, ct, et, dt_)


def kernel(temporal_pos, causal_depth, epistemic_pos, deontic_pos,
           temporal_table, causal_table, epistemic_table, deontic_table):
    b, s = temporal_pos.shape
    out = _lookup(temporal_pos, causal_depth, epistemic_pos, deontic_pos,
                  temporal_table, causal_table, epistemic_table, deontic_table)
    return out.reshape(b, s, D_MODEL)


# speedup vs baseline: 2.3874x; 1.0268x over previous
import jax
import jax.numpy as jnp
from jax import lax
from jax.experimental import pallas as pl
from jax.experimental.pallas import tpu as pltpu
from jax.experimental.pallas import tpu_sc as plsc

D_SUB = 256
D_MODEL = 1024
NC, NS = 2, 16
NW = NC * NS
B_TOTAL = 4 * 8192
PER_W = B_TOTAL // NW
C = 64
NCHUNK = PER_W // C


def _sc_body(tpos, cpos, epos, dpos, tt, ct, et, dt_, out,
             ia0, ia1, ia2, ia3, b0, b1, b2, b3,
             g0, g1, g2, g3, w0, w1, w2, w3):
    wid = lax.axis_index("s") * NC + lax.axis_index("c")
    base_w = wid * PER_W
    pos_refs = (tpos, cpos, epos, dpos)
    tab_refs = (tt, ct, et, dt_)
    idx_all = (ia0, ia1, ia2, ia3)
    bufs = (b0, b1, b2, b3)
    gsems = (g0, g1, g2, g3)
    wsems = (w0, w1, w2, w3)

    for t in range(4):
        pltpu.sync_copy(pos_refs[t].at[pl.ds(base_w, PER_W)], idx_all[t])

    def gather(chunk, t):
        return pltpu.make_async_copy(
            tab_refs[t].at[idx_all[t].at[pl.ds(chunk * C, C)]],
            bufs[t], gsems[t])

    def write(chunk, t):
        return pltpu.make_async_copy(
            bufs[t],
            out.at[pl.ds(base_w + chunk * C, C), pl.ds(t * D_SUB, D_SUB)],
            wsems[t])

    for t in range(4):
        gather(0, t).start()

    @pl.loop(0, NCHUNK - 1)
    def _steady(chunk):
        for t in range(4):
            gather(chunk, t).wait()
            wd = write(chunk, t)
            wd.start()
            wd.wait()
            gather(chunk + 1, t).start()

    for t in range(4):
        gather(NCHUNK - 1, t).wait()
        wd = write(NCHUNK - 1, t)
        wd.start()
        wd.wait()


@jax.jit
def _lookup(tpos, cpos, epos, dpos, tt, ct, et, dt_):
    mesh = plsc.VectorSubcoreMesh(
        core_axis_name="c", subcore_axis_name="s",
        num_cores=NC, num_subcores=NS)
    scratch = (
        [pltpu.VMEM((PER_W,), jnp.int32) for _ in range(4)]
        + [pltpu.VMEM((C, D_SUB), jnp.float32) for _ in range(4)]
        + [pltpu.SemaphoreType.DMA for _ in range(8)]
    )
    f = pl.kernel(
        _sc_body,
        out_type=jax.ShapeDtypeStruct((B_TOTAL, D_MODEL), jnp.float32),
        mesh=mesh,
        scratch_types=scratch,
    )
    return f(tpos.reshape(-1), cpos.reshape(-1), epos.reshape(-1),
             dpos.reshape(-1), tt, ct, et, dt_)


def kernel(temporal_pos, causal_depth, epistemic_pos, deontic_pos,
           temporal_table, causal_table, epistemic_table, deontic_table):
    b, s = temporal_pos.shape
    out = _lookup(temporal_pos, causal_depth, epistemic_pos, deontic_pos,
                  temporal_table, causal_table, epistemic_table, deontic_table)
    return out.reshape(b, s, D_MODEL)
